# Initial kernel scaffold; baseline (speedup 1.0000x reference)
import jax
import jax.numpy as jnp
from jax.experimental import pallas as pl


def kernel(x, edge_index):
    raise NotImplementedError("write your pallas kernel here")


# speedup vs baseline: 18.8926x; 18.8926x over previous
import functools

import jax
import jax.numpy as jnp
from jax import lax
from jax.experimental import pallas as pl
from jax.experimental.pallas import tpu as pltpu
from jax.experimental.pallas import tpu_sc as plsc

N_NODES = 10000
N_EDGES = 320000
D_FEAT = 128

NC = 2
NS = 16
NW = NC * NS
CH = 128
CHUNKS_PER_W = 80
E_PAD = NW * CHUNKS_PER_W * CH
N_PAD = 10240
STRIPE = N_PAD // NS
DUMMY = N_NODES


def _deg_body(row_hbm, col_hbm, out_hbm, rbuf, cbuf, mbuf, vbuf, zbuf, cnt_sh):
    c = lax.axis_index("c")
    s = lax.axis_index("s")
    w = s * NC + c

    def _z(i, carry):
        zbuf[pl.ds(i * 16, 16)] = jnp.zeros((16,), jnp.float32)
        return carry
    lax.fori_loop(jnp.int32(0), jnp.int32(STRIPE // 16), _z, jnp.int32(0))
    pltpu.sync_copy(zbuf, cnt_sh.at[pl.ds(s * STRIPE, STRIPE)])

    for i in range(CH // 16):
        vbuf[pl.ds(i * 16, 16)] = jnp.ones((16,), jnp.float32)

    nloc = CHUNKS_PER_W * CH
    pltpu.sync_copy(row_hbm.at[pl.ds(w * nloc, nloc)], rbuf)
    pltpu.sync_copy(col_hbm.at[pl.ds(w * nloc, nloc)], cbuf)

    plsc.subcore_barrier()

    def _step(k, carry):
        base = k * CH
        for i in range(CH // 16):
            r = rbuf[pl.ds(base + i * 16, 16)]
            cc = cbuf[pl.ds(base + i * 16, 16)]
            mbuf[pl.ds(i * 16, 16)] = jnp.where(
                r == cc, jnp.full((16,), DUMMY, jnp.int32), cc)
        pltpu.sync_copy(vbuf, cnt_sh.at[mbuf], add=True)
        return carry
    lax.fori_loop(jnp.int32(0), jnp.int32(CHUNKS_PER_W), _step, jnp.int32(0))

    plsc.subcore_barrier()
    pltpu.sync_copy(cnt_sh.at[pl.ds(s * STRIPE, STRIPE)],
                    out_hbm.at[c].at[pl.ds(s * STRIPE, STRIPE)])


_deg_kernel = functools.partial(
    pl.kernel,
    out_type=jax.ShapeDtypeStruct((NC, N_PAD), jnp.float32),
    mesh=plsc.VectorSubcoreMesh(core_axis_name="c", subcore_axis_name="s"),
    scratch_types=[
        pltpu.VMEM((CHUNKS_PER_W * CH,), jnp.int32),
        pltpu.VMEM((CHUNKS_PER_W * CH,), jnp.int32),
        pltpu.VMEM((CH,), jnp.int32),
        pltpu.VMEM((CH,), jnp.float32),
        pltpu.VMEM((STRIPE,), jnp.float32),
        pltpu.VMEM_SHARED((N_PAD,), jnp.float32),
    ],
)(_deg_body)


DH = D_FEAT // 2
CHUNKS_B = E_PAD // (NS * CH)


def _prop_body(row_hbm, col_hbm, y_hbm, out_hbm,
               rbuf, cbuf, mbuf, rowa, rowb, zbuf, acc_sh, sema, semb):
    c = lax.axis_index("c")
    s = lax.axis_index("s")

    for r in range(16):
        for j in range(DH // 16):
            zbuf[r, pl.ds(j * 16, 16)] = jnp.zeros((16,), jnp.float32)

    def _z(i, carry):
        pltpu.sync_copy(zbuf, acc_sh.at[pl.ds(s * STRIPE + i * 16, 16)])
        return carry
    lax.fori_loop(jnp.int32(0), jnp.int32(STRIPE // 16), _z, jnp.int32(0))

    nloc = CHUNKS_B * CH
    pltpu.sync_copy(row_hbm.at[pl.ds(s * nloc, nloc)], rbuf)
    pltpu.sync_copy(col_hbm.at[pl.ds(s * nloc, nloc)], cbuf)

    plsc.subcore_barrier()

    def _mask_cols(k):
        base = k * CH
        for i in range(CH // 16):
            r = rbuf[pl.ds(base + i * 16, 16)]
            cc = cbuf[pl.ds(base + i * 16, 16)]
            mbuf[pl.ds(i * 16, 16)] = jnp.where(
                r == cc, jnp.full((16,), DUMMY, jnp.int32), cc)

    def _gather_start(k, buf, sem):
        pltpu.async_copy(y_hbm.at[c].at[rbuf.at[pl.ds(k * CH, CH)]], buf, sem)

    def _gather_wait(k, buf, sem):
        pltpu.make_async_copy(
            y_hbm.at[c].at[rbuf.at[pl.ds(k * CH, CH)]], buf, sem).wait()

    _gather_start(0, rowa, sema)

    def _step(j, carry):
        k0 = 2 * j
        _gather_start(k0 + 1, rowb, semb)
        _gather_wait(k0, rowa, sema)
        _mask_cols(k0)
        pltpu.sync_copy(rowa, acc_sh.at[mbuf], add=True)

        @pl.when(j < CHUNKS_B // 2 - 1)
        def _():
            _gather_start(k0 + 2, rowa, sema)

        _gather_wait(k0 + 1, rowb, semb)
        _mask_cols(k0 + 1)
        pltpu.sync_copy(rowb, acc_sh.at[mbuf], add=True)
        return carry
    lax.fori_loop(jnp.int32(0), jnp.int32(CHUNKS_B // 2), _step, jnp.int32(0))

    plsc.subcore_barrier()

    def _out(i, carry):
        pltpu.sync_copy(acc_sh.at[pl.ds(s * STRIPE + i * CH, CH)], rowa)
        pltpu.sync_copy(rowa, out_hbm.at[c].at[pl.ds(s * STRIPE + i * CH, CH)])
        return carry
    lax.fori_loop(jnp.int32(0), jnp.int32(STRIPE // CH), _out, jnp.int32(0))


_prop_kernel = functools.partial(
    pl.kernel,
    out_type=jax.ShapeDtypeStruct((NC, N_PAD, DH), jnp.float32),
    mesh=plsc.VectorSubcoreMesh(core_axis_name="c", subcore_axis_name="s"),
    scratch_types=[
        pltpu.VMEM((CHUNKS_B * CH,), jnp.int32),
        pltpu.VMEM((CHUNKS_B * CH,), jnp.int32),
        pltpu.VMEM((CH,), jnp.int32),
        pltpu.VMEM((CH, DH), jnp.float32),
        pltpu.VMEM((CH, DH), jnp.float32),
        pltpu.VMEM((16, DH), jnp.float32),
        pltpu.VMEM_SHARED((N_PAD, DH), jnp.float32),
        pltpu.SemaphoreType.DMA,
        pltpu.SemaphoreType.DMA,
    ],
    compiler_params=pltpu.CompilerParams(use_tc_tiling_on_sc=False),
)(_prop_body)


def _e1_body(cnt_ref, dinv_ref):
    deg = cnt_ref[0] + cnt_ref[1] + 1.0
    dinv_ref[...] = lax.rsqrt(deg)


def _e2_body(x_ref, dinv_ref, y_ref):
    d = dinv_ref[...]
    y_ref[0] = x_ref[:, 0:64] * d
    y_ref[1] = x_ref[:, 64:128] * d


def _e3_body(p_ref, x_ref, dinv_ref, out_ref):
    dinv = dinv_ref[...]
    x = x_ref[...]
    out_ref[:, 0:64] = dinv * p_ref[0] + dinv * dinv * x[:, 0:64]
    out_ref[:, 64:128] = dinv * p_ref[1] + dinv * dinv * x[:, 64:128]


def kernel(x, edge_index):
    ei = edge_index.astype(jnp.int32)
    row, col = ei[0], ei[1]
    pad = E_PAD - N_EDGES
    row = jnp.concatenate([row, jnp.zeros((pad,), jnp.int32)])
    col = jnp.concatenate([col, jnp.full((pad,), DUMMY, jnp.int32)])

    cnt = _deg_kernel(row, col)

    dinv3 = pl.pallas_call(
        _e1_body,
        out_shape=jax.ShapeDtypeStruct((N_PAD // 128, 128), jnp.float32),
    )(cnt.reshape(NC, N_PAD // 128, 128))
    dinv_col = dinv3.reshape(N_PAD)[:N_NODES, None]

    y = pl.pallas_call(
        _e2_body,
        out_shape=jax.ShapeDtypeStruct((NC, N_NODES, 64), jnp.float32),
    )(x, dinv_col)

    p = _prop_kernel(row, col, y)

    out = pl.pallas_call(
        _e3_body,
        out_shape=jax.ShapeDtypeStruct((N_NODES, D_FEAT), jnp.float32),
    )(p[:, :N_NODES, :], x, dinv_col)
    return out
